# baseline (speedup 1.0000x reference)
import functools

import jax
import jax.numpy as jnp
from jax import lax
from jax.experimental import pallas as pl
from jax.experimental.pallas import tpu as pltpu
from jax.experimental.pallas import tpu_sc as plsc

_IDX_ROWS = ((2, 7, 0), (5, 6, 3), (4, 0, 5), (1, 5, 6))
_FLAT_IDX = tuple(
    r * 8 + c for r, row in enumerate(_IDX_ROWS) for c in row
) + (0, 0, 0, 0)


def _body(x_hbm, idx_hbm, out_hbm, x_v, idx_v, out_v):
    wid = lax.axis_index("c") * 16 + lax.axis_index("s")

    @pl.when(wid == 0)
    def _():
        pltpu.sync_copy(x_hbm, x_v)
        pltpu.sync_copy(idx_hbm, idx_v)
        out_v[...] = plsc.load_gather(x_v, [idx_v[...]])
        pltpu.sync_copy(out_v, out_hbm)


@jax.jit
def _sc_gather(x_flat, idx):
    mesh = plsc.VectorSubcoreMesh(
        core_axis_name="c", subcore_axis_name="s", num_cores=2, num_subcores=16
    )
    return pl.kernel(
        _body,
        out_type=jax.ShapeDtypeStruct((16,), jnp.float32),
        mesh=mesh,
        scratch_types=[
            pltpu.VMEM((32,), jnp.float32),
            pltpu.VMEM((16,), jnp.int32),
            pltpu.VMEM((16,), jnp.float32),
        ],
        compiler_params=pltpu.CompilerParams(needs_layout_passes=False),
    )(x_flat, idx)


def kernel(x):
    x_flat = x.reshape(32)
    idx = jnp.asarray(_FLAT_IDX, dtype=jnp.int32)
    out16 = _sc_gather(x_flat, idx)
    return out16[:12].reshape(4, 3)


# speedup vs baseline: 1.0941x; 1.0941x over previous
import functools

import jax
import jax.numpy as jnp
from jax import lax
from jax.experimental import pallas as pl
from jax.experimental.pallas import tpu as pltpu
from jax.experimental.pallas import tpu_sc as plsc

_IDX_ROWS = ((2, 7, 0), (5, 6, 3), (4, 0, 5), (1, 5, 6))
_FLAT_IDX = tuple(
    r * 8 + c for r, row in enumerate(_IDX_ROWS) for c in row
) + (0, 0, 0, 0)


def _flat_idx_vector():
    lane = lax.iota(jnp.int32, 16)
    idx = jnp.where(lane == 0, _FLAT_IDX[0], 0)
    for k in range(1, 12):
        idx = jnp.where(lane == k, _FLAT_IDX[k], idx)
    return idx


def _body(x_hbm, out_hbm, x_v, out_v):
    wid = lax.axis_index("c") * 16 + lax.axis_index("s")

    @pl.when(wid == 0)
    def _():
        pltpu.sync_copy(x_hbm, x_v)
        out_v[...] = plsc.load_gather(x_v, [_flat_idx_vector()])
        pltpu.sync_copy(out_v, out_hbm)


@jax.jit
def _sc_gather(x_flat):
    mesh = plsc.VectorSubcoreMesh(
        core_axis_name="c", subcore_axis_name="s", num_cores=1, num_subcores=1
    )
    return pl.kernel(
        _body,
        out_type=jax.ShapeDtypeStruct((16,), jnp.float32),
        mesh=mesh,
        scratch_types=[
            pltpu.VMEM((32,), jnp.float32),
            pltpu.VMEM((16,), jnp.float32),
        ],
        compiler_params=pltpu.CompilerParams(needs_layout_passes=False),
    )(x_flat)


def kernel(x):
    out16 = _sc_gather(x.reshape(32))
    return out16[:12].reshape(4, 3)


# speedup vs baseline: 1.1712x; 1.0704x over previous
import functools

import jax
import jax.numpy as jnp
from jax import lax
from jax.experimental import pallas as pl
from jax.experimental.pallas import tpu as pltpu
from jax.experimental.pallas import tpu_sc as plsc

_COLS = (2, 7, 0, 5, 6, 3, 4, 0, 5, 1, 5, 6)


def _body(x_hbm, out_hbm, x_v, out_v):
    lane = lax.iota(jnp.int32, 16)
    row = lane // 3
    out_col = lane % 3
    src_col = jnp.where(lane == 0, _COLS[0], 0)
    for m in range(1, 12):
        src_col = jnp.where(lane == m, _COLS[m], src_col)
    valid = lane < 12
    row = jnp.where(valid, row, 0)

    pltpu.sync_copy(x_hbm, x_v)
    vals = plsc.load_gather(x_v, [row, src_col])
    plsc.store_scatter(out_v, [row, out_col], vals, mask=valid)
    pltpu.sync_copy(out_v, out_hbm)


@jax.jit
def kernel(x):
    mesh = plsc.VectorSubcoreMesh(
        core_axis_name="c", subcore_axis_name="s", num_cores=1, num_subcores=1
    )
    return pl.kernel(
        _body,
        out_type=jax.ShapeDtypeStruct((4, 3), jnp.float32),
        mesh=mesh,
        scratch_types=[
            pltpu.VMEM((4, 8), jnp.float32),
            pltpu.VMEM((4, 3), jnp.float32),
        ],
        compiler_params=pltpu.CompilerParams(needs_layout_passes=False),
    )(x)


# speedup vs baseline: 1.2710x; 1.0853x over previous
import jax
import jax.numpy as jnp
from jax.experimental import pallas as pl
from jax.experimental.pallas import tpu as pltpu
from jax.experimental.pallas import tpu_sc as plsc

_IDX_ROWS = ((2, 7, 0), (5, 6, 3), (4, 0, 5), (1, 5, 6))


def _body(x_hbm, out_hbm, x_s, out_s):
    pltpu.sync_copy(x_hbm, x_s)
    for r, row in enumerate(_IDX_ROWS):
        for j, c in enumerate(row):
            out_s[r, j] = x_s[r, c]
    pltpu.sync_copy(out_s, out_hbm)


@jax.jit
def kernel(x):
    mesh = plsc.ScalarSubcoreMesh(axis_name="c", num_cores=1)
    return pl.kernel(
        _body,
        out_type=jax.ShapeDtypeStruct((4, 3), jnp.float32),
        mesh=mesh,
        scratch_types=[
            pltpu.SMEM((4, 8), jnp.float32),
            pltpu.SMEM((4, 3), jnp.float32),
        ],
        compiler_params=pltpu.CompilerParams(needs_layout_passes=False),
    )(x)
